# baseline (speedup 1.0000x reference)
import functools

import jax
import jax.numpy as jnp
from jax import lax
from jax.experimental import pallas as pl
from jax.experimental.pallas import tpu as pltpu
from jax.experimental.pallas import tpu_sc as plsc

B = 4096
H = 200
D = 32
H2 = 2 * H
GSZ = (104, 104, 104, 88)
NBUF = 4
RPI = 20
NACC = 4

_info = plsc.get_sparse_core_info()
NC, NS, L = _info.num_cores, _info.num_subcores, _info.num_lanes
NW = NC * NS
BPW = B // NW
IPW = BPW * H
OPW = BPW * D

_mesh = plsc.VectorSubcoreMesh(core_axis_name="c", subcore_axis_name="s")


@functools.partial(
    pl.kernel,
    mesh=_mesh,
    out_type=jax.ShapeDtypeStruct((B * D,), jnp.float32),
    compiler_params=pltpu.CompilerParams(
        use_tc_tiling_on_sc=False, needs_layout_passes=False
    ),
    scratch_types=[
        pltpu.VMEM((IPW,), jnp.int32),
        pltpu.VMEM((BPW * H2,), jnp.int32),
        [pltpu.VMEM((H2, L), jnp.float32) for _ in range(NBUF)],
        pltpu.VMEM((OPW,), jnp.float32),
        [pltpu.SemaphoreType.DMA for _ in range(NBUF)],
    ],
)
def _encode(x_hbm, table_hbm, out_hbm, idx_v, idx2_v, rows, out_v, sems):
    wid = lax.axis_index("s") * NC + lax.axis_index("c")

    pltpu.sync_copy(x_hbm.at[pl.ds(wid * IPW, IPW)], idx_v)

    lane = lax.iota(jnp.int32, L)
    even = 2 * lane
    odd = even + 1

    def expand(g, _):
        v = idx_v[pl.ds(g * L, L)]
        a = v + v
        base = 2 * g * L
        plsc.store_scatter(idx2_v, [base + even], a)
        plsc.store_scatter(idx2_v, [base + odd], a + 1)
        return 0

    lax.fori_loop(0, IPW // L, expand, 0)

    def start_gather(i, b):
        off = 0
        for g in GSZ:
            pltpu.async_copy(
                table_hbm.at[idx2_v.at[pl.ds(i * H2 + off, g)]],
                rows[b].at[pl.ds(off, g)],
                sems[b],
            )
            off += g

    def wait_gather(i, b):
        off = 0
        for g in GSZ:
            pltpu.make_async_copy(
                table_hbm.at[idx2_v.at[pl.ds(i * H2 + off, g)]],
                rows[b].at[pl.ds(off, g)],
                sems[b],
            ).wait()
            off += g

    def reduce_row(i, buf):
        zero = jnp.zeros((L,), jnp.float32)

        def body(j, accs):
            accs = list(accs)
            for r in range(RPI):
                row = RPI * j + r
                lo, hi = accs[r % NACC]
                lo = lo + buf[2 * row, pl.ds(0, L)]
                hi = hi + buf[2 * row + 1, pl.ds(0, L)]
                accs[r % NACC] = (lo, hi)
            return tuple(accs)

        accs = lax.fori_loop(0, H // RPI, body, tuple((zero, zero) for _ in range(NACC)))
        lo = accs[0][0] + accs[1][0] + accs[2][0] + accs[3][0]
        hi = accs[0][1] + accs[1][1] + accs[2][1] + accs[3][1]
        scale = jnp.float32(1.0 / H)
        out_v[pl.ds(i * D, L)] = lo * scale
        out_v[pl.ds(i * D + L, L)] = hi * scale

    for b in range(NBUF):
        start_gather(b, b)

    def outer(k, _):
        i0 = NBUF * k
        for b in range(NBUF):
            wait_gather(i0 + b, b)
            reduce_row(i0 + b, rows[b])
            start_gather(i0 + b + NBUF, b)
        return 0

    lax.fori_loop(0, BPW // NBUF - 1, outer, 0)

    for b in range(NBUF):
        i = BPW - NBUF + b
        wait_gather(i, b)
        reduce_row(i, rows[b])

    pltpu.sync_copy(out_v, out_hbm.at[pl.ds(wid * OPW, OPW)])


V = 1000000
FCH = 248
NCH = 126
GBASE = 3906


@functools.partial(
    pl.kernel,
    mesh=_mesh,
    out_type=jax.ShapeDtypeStruct((V * D,), jnp.float32),
    compiler_params=pltpu.CompilerParams(use_tc_tiling_on_sc=True),
    scratch_types=[
        [pltpu.VMEM((FCH, D), jnp.float32) for _ in range(2)],
        [pltpu.VMEM((FCH * D,), jnp.float32) for _ in range(2)],
        [pltpu.SemaphoreType.DMA for _ in range(2)],
        [pltpu.SemaphoreType.DMA for _ in range(2)],
    ],
)
def _relayout(table_hbm, tf_hbm, vbufs, fbufs, isems, osems):
    wid = lax.axis_index("s") * NC + lax.axis_index("c")
    r0 = 8 * (wid * GBASE + jnp.minimum(wid, 8))

    def start_in(c, b):
        pltpu.async_copy(table_hbm.at[pl.ds(r0 + c * FCH, FCH)], vbufs[b], isems[b])

    def wait_in(c, b):
        pltpu.make_async_copy(
            table_hbm.at[pl.ds(r0 + c * FCH, FCH)], vbufs[b], isems[b]
        ).wait()

    def detile(b):
        def body(j, _):
            for r in range(8):
                g = 8 * j + r
                fbufs[b][pl.ds(g * D, L)] = vbufs[b][g, pl.ds(0, L)]
                fbufs[b][pl.ds(g * D + L, L)] = vbufs[b][g, pl.ds(L, L)]
            return 0

        lax.fori_loop(0, FCH // 8, body, 0)

    def start_out(c, b):
        pltpu.async_copy(
            fbufs[b], tf_hbm.at[pl.ds((r0 + c * FCH) * D, FCH * D)], osems[b]
        )

    def wait_out(c, b):
        pltpu.make_async_copy(
            fbufs[b], tf_hbm.at[pl.ds((r0 + c * FCH) * D, FCH * D)], osems[b]
        ).wait()

    start_in(0, 0)
    start_in(1, 1)

    def chunk(c, _):
        b = 0
        wait_in(c, b)
        detile(b)
        start_in(c + 2, b)
        start_out(c, b)
        wait_out(c, b)
        b = 1
        wait_in(c + 1, b)
        detile(b)
        start_in(c + 3, b)
        start_out(c + 1, b)
        wait_out(c + 1, b)
        return 0

    lax.fori_loop(0, NCH // 2 - 1, chunk, 0)

    for c, b in ((NCH - 2, 0), (NCH - 1, 1)):
        wait_in(c, b)
        detile(b)
        start_out(c, b)
        wait_out(c, b)

    @pl.when(wid < 8)
    def _():
        rx = r0 + NCH * FCH
        pltpu.sync_copy(table_hbm.at[pl.ds(rx, 8)], vbufs[0].at[pl.ds(0, 8)])
        for g in range(8):
            fbufs[0][pl.ds(g * D, L)] = vbufs[0][g, pl.ds(0, L)]
            fbufs[0][pl.ds(g * D + L, L)] = vbufs[0][g, pl.ds(L, L)]
        pltpu.sync_copy(
            fbufs[0].at[pl.ds(0, 8 * D)], tf_hbm.at[pl.ds(rx * D, 8 * D)]
        )


def kernel(x, table):
    tf = _relayout(table)
    flat = _encode(x.astype(jnp.int32).reshape(B * H), tf.reshape(2 * V, L))
    return flat.reshape(B, D)


# speedup vs baseline: 1.1801x; 1.1801x over previous
import functools

import jax
import jax.numpy as jnp
from jax import lax
from jax.experimental import pallas as pl
from jax.experimental.pallas import tpu as pltpu
from jax.experimental.pallas import tpu_sc as plsc

B = 4096
H = 200
D = 32
GA = 104
GB = H - GA
NBUF = 8
RPI = 20
NACC = 4

_info = plsc.get_sparse_core_info()
NC, NS, L = _info.num_cores, _info.num_subcores, _info.num_lanes
NW = NC * NS
BPW = B // NW
IPW = BPW * H
OPW = BPW * D

_mesh = plsc.VectorSubcoreMesh(core_axis_name="c", subcore_axis_name="s")


@functools.partial(
    pl.kernel,
    mesh=_mesh,
    out_type=jax.ShapeDtypeStruct((B * D,), jnp.float32),
    compiler_params=pltpu.CompilerParams(use_tc_tiling_on_sc=False),
    scratch_types=[
        pltpu.VMEM((IPW,), jnp.int32),
        [pltpu.VMEM((H, D), jnp.float32) for _ in range(NBUF)],
        pltpu.VMEM((OPW,), jnp.float32),
        [pltpu.SemaphoreType.DMA for _ in range(NBUF)],
    ],
)
def _encode(x_hbm, table_hbm, out_hbm, idx_v, rows, out_v, sems):
    wid = lax.axis_index("s") * NC + lax.axis_index("c")

    pltpu.sync_copy(x_hbm.at[pl.ds(wid * IPW, IPW)], idx_v)

    def start_gather(i, b):
        pltpu.async_copy(
            table_hbm.at[idx_v.at[pl.ds(i * H, GA)]], rows[b].at[pl.ds(0, GA)], sems[b]
        )
        pltpu.async_copy(
            table_hbm.at[idx_v.at[pl.ds(i * H + GA, GB)]],
            rows[b].at[pl.ds(GA, GB)],
            sems[b],
        )

    def wait_gather(i, b):
        pltpu.make_async_copy(
            table_hbm.at[idx_v.at[pl.ds(i * H, GA)]], rows[b].at[pl.ds(0, GA)], sems[b]
        ).wait()
        pltpu.make_async_copy(
            table_hbm.at[idx_v.at[pl.ds(i * H + GA, GB)]],
            rows[b].at[pl.ds(GA, GB)],
            sems[b],
        ).wait()

    def reduce_row(i, buf):
        zero = jnp.zeros((L,), jnp.float32)

        def body(j, accs):
            accs = list(accs)
            for r in range(RPI):
                row = RPI * j + r
                lo, hi = accs[r % NACC]
                lo = lo + buf[row, pl.ds(0, L)]
                hi = hi + buf[row, pl.ds(L, L)]
                accs[r % NACC] = (lo, hi)
            return tuple(accs)

        accs = lax.fori_loop(0, H // RPI, body, tuple((zero, zero) for _ in range(NACC)))
        lo = accs[0][0] + accs[1][0] + accs[2][0] + accs[3][0]
        hi = accs[0][1] + accs[1][1] + accs[2][1] + accs[3][1]
        scale = jnp.float32(1.0 / H)
        out_v[pl.ds(i * D, L)] = lo * scale
        out_v[pl.ds(i * D + L, L)] = hi * scale

    for b in range(NBUF):
        start_gather(b, b)

    def outer(k, _):
        i0 = NBUF * k
        for b in range(NBUF):
            wait_gather(i0 + b, b)
            reduce_row(i0 + b, rows[b])
            start_gather(i0 + b + NBUF, b)
        return 0

    lax.fori_loop(0, BPW // NBUF - 1, outer, 0)

    for b in range(NBUF):
        i = BPW - NBUF + b
        wait_gather(i, b)
        reduce_row(i, rows[b])

    pltpu.sync_copy(out_v, out_hbm.at[pl.ds(wid * OPW, OPW)])


def kernel(x, table):
    flat = _encode(x.astype(jnp.int32).reshape(B * H), table)
    return flat.reshape(B, D)
